# baseline (speedup 1.0000x reference)
import functools

import jax
import jax.numpy as jnp
from jax import lax
from jax.experimental import pallas as pl
from jax.experimental.pallas import tpu as pltpu
from jax.experimental.pallas import tpu_sc as plsc

BATCH = 16384
D = 32
NC = 2
NS = 16
L = 16
NW = NC * NS
B_PER_W = BATCH // NW
CHUNK = 128
N_CHUNK = B_PER_W // CHUNK
N_GROUP = B_PER_W // L


def _bpr_body(uidx_hbm, iidx_hbm, utab_hbm, itab_hbm, out_hbm,
              uidx_v, iidx_v, urows_v, irows_v, out_v, sem_u, sem_i):
    c = lax.axis_index("c")
    s = lax.axis_index("s")
    wid = s * NC + c

    pltpu.sync_copy(uidx_hbm.at[pl.ds(wid * N_CHUNK, N_CHUNK)], uidx_v)
    pltpu.sync_copy(iidx_hbm.at[pl.ds(wid * N_CHUNK, N_CHUNK)], iidx_v)

    copies = []
    for j in range(N_CHUNK):
        copies.append(pltpu.async_copy(
            utab_hbm.at[uidx_v.at[j]], urows_v.at[pl.ds(j * CHUNK, CHUNK)], sem_u))
        copies.append(pltpu.async_copy(
            itab_hbm.at[iidx_v.at[j]], irows_v.at[pl.ds(j * CHUNK, CHUNK)], sem_i))
    for cp in copies:
        cp.wait()

    lane = jnp.arange(L, dtype=jnp.int32)

    def group_body(g, carry):
        acc = jnp.zeros((L,), jnp.float32)
        for l in range(L):
            r = g * L + l
            u0 = urows_v[r, pl.ds(0, L)]
            u1 = urows_v[r, pl.ds(L, L)]
            i0 = irows_v[r, pl.ds(0, L)]
            i1 = irows_v[r, pl.ds(L, L)]
            s = jnp.sum(u0 * i0 + u1 * i1)
            acc = jnp.where(lane == l, s, acc)
        out_v[pl.ds(g * L, L)] = acc
        return carry

    lax.fori_loop(0, N_GROUP, group_body, 0)

    pltpu.sync_copy(out_v, out_hbm.at[pl.ds(wid * B_PER_W, B_PER_W)])


_bpr_sc = functools.partial(
    pl.kernel,
    mesh=plsc.VectorSubcoreMesh(core_axis_name="c", subcore_axis_name="s"),
    out_type=jax.ShapeDtypeStruct((BATCH,), jnp.float32),
    compiler_params=pltpu.CompilerParams(
        needs_layout_passes=False, use_tc_tiling_on_sc=False),
    scratch_types=[
        pltpu.VMEM((N_CHUNK, CHUNK), jnp.int32),
        pltpu.VMEM((N_CHUNK, CHUNK), jnp.int32),
        pltpu.VMEM((B_PER_W, D), jnp.float32),
        pltpu.VMEM((B_PER_W, D), jnp.float32),
        pltpu.VMEM((B_PER_W,), jnp.float32),
        pltpu.SemaphoreType.DMA,
        pltpu.SemaphoreType.DMA,
    ],
)(_bpr_body)


@jax.jit
def kernel(user_indices, item_indices, user_table, item_table):
    uidx = user_indices.reshape(BATCH // CHUNK, CHUNK)
    iidx = item_indices.reshape(BATCH // CHUNK, CHUNK)
    return _bpr_sc(uidx, iidx, user_table, item_table)
